# baseline (speedup 1.0000x reference)
import jax
import jax.numpy as jnp
from jax.experimental import pallas as pl
from jax.experimental.pallas import tpu as pltpu


def _make_kernel(B, S, F, E, C, nrows):
    def _fused_sage_kernel(*refs):
        x_refs = refs[:nrows]
        (w1_ref, b1_ref, w2_ref, b2_ref, w_ref,
         lab_ref, loss_ref, preds_ref, labout_ref, h1_ref) = refs[nrows:]
        w1 = w1_ref[...]
        b1 = b1_ref[...]
        for i in range(nrows):
            n_i = min(S, B - i * S)
            xi = x_refs[i][...]
            ci = jax.lax.dot_general(xi, w1, (((1,), (1,)), ((), ())),
                                     preferred_element_type=jnp.float32)
            ci = jnp.maximum(ci + b1, 0.0)
            h1_ref[pl.ds(i * S, n_i), :] = ci[:n_i]
        h1 = h1_ref[...]
        h2 = jax.lax.dot_general(h1, w2_ref[...], (((1,), (1,)), ((), ())),
                                 preferred_element_type=jnp.float32)
        h2 = jnp.maximum(h2 + b2_ref[...], 0.0)
        sT = jax.lax.dot_general(w_ref[...], h2, (((0,), (1,)), ((), ())),
                                 preferred_element_type=jnp.float32)

        lab = lab_ref[...].reshape(1, B)
        m = jnp.max(sT, axis=0, keepdims=True)
        lse = jnp.log(jnp.sum(jnp.exp(sT - m), axis=0, keepdims=True)) + m
        iota = jax.lax.broadcasted_iota(jnp.int32, (C, B), 0)
        picked = jnp.sum(jnp.where(iota == lab, sT, 0.0),
                         axis=0, keepdims=True)
        loss_ref[...] = -jnp.sum(picked - lse, keepdims=True) / B
        preds_ref[...] = jnp.min(jnp.where(sT == m, iota, C), axis=0)
        labout_ref[...] = lab_ref[...]

    return _fused_sage_kernel


def kernel(x0, label, out_1, out_2, W1_l, b1, W1_r, W2_l, b2, W2_r, weight):
    B, S, F = x0.shape
    E = W1_r.shape[0]
    C = weight.shape[1]
    nrows = -(-B // S)
    x_parts = [x0[i] for i in range(nrows)]
    lab = label.astype(jnp.int32)

    loss2d, preds, labels = pl.pallas_call(
        _make_kernel(B, S, F, E, C, nrows),
        out_shape=(
            jax.ShapeDtypeStruct((1, 1), jnp.float32),
            jax.ShapeDtypeStruct((B,), jnp.int32),
            jax.ShapeDtypeStruct((B,), jnp.int32),
        ),
        scratch_shapes=[pltpu.VMEM((B, E), jnp.float32)],
    )(*x_parts, W1_r, b1.reshape(1, E), W2_r, b2.reshape(1, E), weight, lab)

    return loss2d.reshape(()), preds, labels


# speedup vs baseline: 1.6099x; 1.6099x over previous
import jax
import jax.numpy as jnp
from jax.experimental import pallas as pl
from jax.experimental.pallas import tpu as pltpu


def _make_kernel(B, S, F, E, C, nrows):
    def _fused_sage_kernel(xs_ref, w1_ref, b1_ref, w2_ref, b2_ref, w_ref,
                           lab_ref, loss_ref, preds_ref, labout_ref, h1_ref):
        w1 = w1_ref[...]
        b1 = b1_ref[...]
        for i in range(nrows):
            n_i = min(S, B - i * S)
            xi = xs_ref[i]
            ci = jax.lax.dot_general(xi, w1, (((1,), (1,)), ((), ())),
                                     preferred_element_type=jnp.float32)
            ci = jnp.maximum(ci + b1, 0.0)
            h1_ref[pl.ds(i * S, n_i), :] = ci[:n_i]
        h1 = h1_ref[...]
        h2 = jax.lax.dot_general(h1, w2_ref[...], (((1,), (1,)), ((), ())),
                                 preferred_element_type=jnp.float32)
        h2 = jnp.maximum(h2 + b2_ref[...], 0.0)
        sT = jax.lax.dot_general(w_ref[...], h2, (((0,), (1,)), ((), ())),
                                 preferred_element_type=jnp.float32)

        lab = lab_ref[...].reshape(1, B)
        m = jnp.max(sT, axis=0, keepdims=True)
        lse = jnp.log(jnp.sum(jnp.exp(sT - m), axis=0, keepdims=True)) + m
        iota = jax.lax.broadcasted_iota(jnp.int32, (C, B), 0)
        picked = jnp.sum(jnp.where(iota == lab, sT, 0.0),
                         axis=0, keepdims=True)
        loss_ref[...] = -jnp.sum(picked - lse, keepdims=True) / B
        preds_ref[...] = jnp.min(jnp.where(sT == m, iota, C), axis=0)
        labout_ref[...] = lab_ref[...]

    return _fused_sage_kernel


def kernel(x0, label, out_1, out_2, W1_l, b1, W1_r, W2_l, b2, W2_r, weight):
    B, S, F = x0.shape
    E = W1_r.shape[0]
    C = weight.shape[1]
    nrows = -(-B // S)
    xs = x0[:nrows]
    lab = label.astype(jnp.int32)

    loss2d, preds, labels = pl.pallas_call(
        _make_kernel(B, S, F, E, C, nrows),
        out_shape=(
            jax.ShapeDtypeStruct((1, 1), jnp.float32),
            jax.ShapeDtypeStruct((B,), jnp.int32),
            jax.ShapeDtypeStruct((B,), jnp.int32),
        ),
        scratch_shapes=[pltpu.VMEM((B, E), jnp.float32)],
    )(xs, W1_r, b1.reshape(1, E), W2_r, b2.reshape(1, E), weight, lab)

    return loss2d.reshape(()), preds, labels
